# baseline (speedup 1.0000x reference)
import functools

import jax
import jax.numpy as jnp
from jax import lax
from jax.experimental import pallas as pl
from jax.experimental.pallas import tpu as pltpu
from jax.experimental.pallas import tpu_sc as plsc


def _make_gather(R0, R1, D, NC, NS):
    B = R0 * R1
    NW = NC * NS
    rows_per_w = R0 // NW
    b_per_w = B // NW
    XR = 8
    CHUNK = XR * R1
    NBUF = 3
    n_chunks = rows_per_w // XR
    assert R0 % NW == 0 and rows_per_w % XR == 0 and n_chunks > NBUF
    assert b_per_w % 16 == 0

    mesh = plsc.VectorSubcoreMesh(core_axis_name="c", subcore_axis_name="s")

    @functools.partial(
        pl.kernel,
        mesh=mesh,
        out_type=jax.ShapeDtypeStruct((R0, R1, D), jnp.float32),
        scratch_types=[
            pltpu.VMEM((b_per_w,), jnp.int32),
            pltpu.VMEM((NBUF, CHUNK, D), jnp.float32),
            pltpu.SemaphoreType.DMA((NBUF,)),
            pltpu.SemaphoreType.DMA((NBUF,)),
        ],
        compiler_params=pltpu.CompilerParams(use_tc_tiling_on_sc=False),
    )
    def gather_kernel(idx_hbm, table_hbm, out_hbm, idx_v, rows_v, gsem, osem):
        wid = lax.axis_index("s") * NC + lax.axis_index("c")
        base = wid * b_per_w
        row_base = wid * rows_per_w
        pltpu.sync_copy(idx_hbm.at[pl.ds(base, b_per_w)], idx_v)

        def dbl(k, carry):
            sl = pl.ds(k * 16, 16)
            idx_v[sl] = idx_v[sl] * 2
            return carry

        lax.fori_loop(0, b_per_w // 16, dbl, 0)

        def g_start(c):
            b = lax.rem(c, NBUF)
            pltpu.async_copy(
                table_hbm.at[idx_v.at[pl.ds(c * CHUNK, CHUNK)]],
                rows_v.at[b],
                gsem.at[b],
            )

        def g_wait(c):
            b = lax.rem(c, NBUF)
            pltpu.make_async_copy(
                table_hbm.at[idx_v.at[pl.ds(c * CHUNK, CHUNK)]],
                rows_v.at[b],
                gsem.at[b],
            ).wait()

        def s_start(c):
            b = lax.rem(c, NBUF)
            for j in range(XR):
                pltpu.async_copy(
                    rows_v.at[b].at[pl.ds(j * R1, R1)],
                    out_hbm.at[row_base + c * XR + j],
                    osem.at[b],
                )

        def s_wait(c):
            b = lax.rem(c, NBUF)
            for j in range(XR):
                pltpu.make_async_copy(
                    rows_v.at[b].at[pl.ds(j * R1, R1)],
                    out_hbm.at[row_base + c * XR + j],
                    osem.at[b],
                ).wait()

        g_start(0)
        g_start(1)

        def body(c, carry):
            g_wait(c)
            s_start(c)
            c2 = c + (NBUF - 1)

            @pl.when(c2 < n_chunks)
            def _():
                @pl.when(c >= 1)
                def _():
                    s_wait(c - 1)

                g_start(c2)

            return carry

        lax.fori_loop(0, n_chunks, body, 0)

        for k in range(NBUF):
            s_wait(n_chunks - NBUF + k)

    return gather_kernel


def kernel(x, pe_weight):
    R0, R1 = x.shape
    V, D = pe_weight.shape
    table2 = jnp.pad(pe_weight, ((0, 0), (0, D))).reshape(2 * V, D)
    info = plsc.get_sparse_core_info()
    half = R1 // 2
    fn = _make_gather(R0, half, D, info.num_cores, info.num_subcores)
    out_a = fn(x[:, :half].reshape(-1), table2)
    out_b = fn(x[:, half:].reshape(-1), table2)
    return jnp.concatenate([out_a, out_b], axis=1)


# speedup vs baseline: 1.0870x; 1.0870x over previous
import functools

import jax
import jax.numpy as jnp
from jax import lax
from jax.experimental import pallas as pl
from jax.experimental.pallas import tpu as pltpu
from jax.experimental.pallas import tpu_sc as plsc


def _make_gather(R0, R1, D, NC, NS):
    B = R0 * R1
    NW = NC * NS
    rows_per_w = R0 // NW
    b_per_w = B // NW
    XR = 8
    CHUNK = XR * R1
    NBUF = 3
    n_chunks = rows_per_w // XR
    assert R0 % NW == 0 and rows_per_w % XR == 0 and n_chunks > NBUF
    assert b_per_w % 16 == 0

    mesh = plsc.VectorSubcoreMesh(core_axis_name="c", subcore_axis_name="s")

    @functools.partial(
        pl.kernel,
        mesh=mesh,
        out_type=jax.ShapeDtypeStruct((R0, R1, D), jnp.float32),
        scratch_types=[
            pltpu.VMEM((b_per_w,), jnp.int32),
            pltpu.VMEM((NBUF, CHUNK, D), jnp.float32),
            pltpu.SemaphoreType.DMA((NBUF,)),
            pltpu.SemaphoreType.DMA((NBUF,)),
        ],
        compiler_params=pltpu.CompilerParams(use_tc_tiling_on_sc=False),
    )
    def gather_kernel(idx_hbm, table_hbm, out_hbm, idx_v, rows_v, gsem, osem):
        wid = lax.axis_index("s") * NC + lax.axis_index("c")
        base = wid * b_per_w
        row_base = wid * rows_per_w
        pltpu.sync_copy(idx_hbm.at[pl.ds(base, b_per_w)], idx_v)

        def dbl(k, carry):
            sl = pl.ds(k * 16, 16)
            idx_v[sl] = idx_v[sl] * 2
            return carry

        lax.fori_loop(0, b_per_w // 16, dbl, 0)

        def g_start(c):
            b = lax.rem(c, NBUF)
            pltpu.async_copy(
                table_hbm.at[idx_v.at[pl.ds(c * CHUNK, CHUNK)]],
                rows_v.at[b],
                gsem.at[b],
            )

        def g_wait(c):
            b = lax.rem(c, NBUF)
            pltpu.make_async_copy(
                table_hbm.at[idx_v.at[pl.ds(c * CHUNK, CHUNK)]],
                rows_v.at[b],
                gsem.at[b],
            ).wait()

        def s_start(c):
            b = lax.rem(c, NBUF)
            for j in range(XR):
                pltpu.async_copy(
                    rows_v.at[b].at[pl.ds(j * R1, R1)],
                    out_hbm.at[row_base + c * XR + j],
                    osem.at[b],
                )

        def s_wait(c):
            b = lax.rem(c, NBUF)
            for j in range(XR):
                pltpu.make_async_copy(
                    rows_v.at[b].at[pl.ds(j * R1, R1)],
                    out_hbm.at[row_base + c * XR + j],
                    osem.at[b],
                ).wait()

        g_start(0)
        g_start(1)

        def body(c, carry):
            g_wait(c)
            s_start(c)
            c2 = c + (NBUF - 1)

            @pl.when(c2 < n_chunks)
            def _():
                @pl.when(c >= 1)
                def _():
                    s_wait(c - 1)

                g_start(c2)

            return carry

        lax.fori_loop(0, n_chunks, body, 0)

        for k in range(NBUF):
            s_wait(n_chunks - NBUF + k)

    return gather_kernel


def kernel(x, pe_weight):
    R0, R1 = x.shape
    V, D = pe_weight.shape
    table2 = jnp.pad(pe_weight, ((0, 0), (0, D))).reshape(2 * V, D)
    info = plsc.get_sparse_core_info()
    fn = _make_gather(R0, R1, D, info.num_cores, info.num_subcores)
    return fn(x.reshape(-1), table2)


# speedup vs baseline: 1.0946x; 1.0069x over previous
import functools

import jax
import jax.numpy as jnp
from jax import lax
from jax.experimental import pallas as pl
from jax.experimental.pallas import tpu as pltpu
from jax.experimental.pallas import tpu_sc as plsc


def _make_gather(R0, R1, D, NC, NS):
    B = R0 * R1
    NW = NC * NS
    rows_per_w = R0 // NW
    b_per_w = B // NW
    XR = 8
    CHUNK = XR * R1
    NBUF = 4
    n_chunks = rows_per_w // XR
    assert R0 % NW == 0 and rows_per_w % XR == 0 and n_chunks > NBUF
    assert b_per_w % 16 == 0

    mesh = plsc.VectorSubcoreMesh(core_axis_name="c", subcore_axis_name="s")

    @functools.partial(
        pl.kernel,
        mesh=mesh,
        out_type=jax.ShapeDtypeStruct((R0, R1, D), jnp.float32),
        scratch_types=[
            pltpu.VMEM((b_per_w,), jnp.int32),
            pltpu.VMEM((NBUF, CHUNK, D), jnp.float32),
            pltpu.SemaphoreType.DMA((NBUF,)),
            pltpu.SemaphoreType.DMA((NBUF,)),
        ],
        compiler_params=pltpu.CompilerParams(use_tc_tiling_on_sc=False),
    )
    def gather_kernel(idx_hbm, table_hbm, out_hbm, idx_v, rows_v, gsem, osem):
        wid = lax.axis_index("s") * NC + lax.axis_index("c")
        base = wid * b_per_w
        row_base = wid * rows_per_w
        pltpu.sync_copy(idx_hbm.at[pl.ds(base, b_per_w)], idx_v)

        def g_start(c):
            b = lax.rem(c, NBUF)
            pltpu.async_copy(
                table_hbm.at[idx_v.at[pl.ds(c * CHUNK, CHUNK)]],
                rows_v.at[b],
                gsem.at[b],
            )

        def g_wait(c):
            b = lax.rem(c, NBUF)
            pltpu.make_async_copy(
                table_hbm.at[idx_v.at[pl.ds(c * CHUNK, CHUNK)]],
                rows_v.at[b],
                gsem.at[b],
            ).wait()

        def s_start(c):
            b = lax.rem(c, NBUF)
            for j in range(XR):
                pltpu.async_copy(
                    rows_v.at[b].at[pl.ds(j * R1, R1)],
                    out_hbm.at[row_base + c * XR + j],
                    osem.at[b],
                )

        def s_wait(c):
            b = lax.rem(c, NBUF)
            for j in range(XR):
                pltpu.make_async_copy(
                    rows_v.at[b].at[pl.ds(j * R1, R1)],
                    out_hbm.at[row_base + c * XR + j],
                    osem.at[b],
                ).wait()

        for p in range(NBUF - 1):
            g_start(p)

        def body(c, carry):
            g_wait(c)
            s_start(c)
            c2 = c + (NBUF - 1)

            @pl.when(c2 < n_chunks)
            def _():
                @pl.when(c >= 1)
                def _():
                    s_wait(c - 1)

                g_start(c2)

            return carry

        lax.fori_loop(0, n_chunks, body, 0)

        for k in range(NBUF):
            s_wait(n_chunks - NBUF + k)

    return gather_kernel


def kernel(x, pe_weight):
    R0, R1 = x.shape
    V, D = pe_weight.shape
    table2 = jnp.pad(pe_weight, ((0, 0), (0, D))).reshape(2 * V, D)
    info = plsc.get_sparse_core_info()
    fn = _make_gather(R0, R1, D, info.num_cores, info.num_subcores)
    return fn(x.reshape(-1) * 2, table2)
